# Initial kernel scaffold; baseline (speedup 1.0000x reference)
import jax
import jax.numpy as jnp
from jax.experimental import pallas as pl


def kernel(x):
    raise NotImplementedError("write your pallas kernel here")


# speedup vs baseline: 23.9795x; 23.9795x over previous
import math

import jax
import jax.numpy as jnp
import numpy as np
from jax import lax
from jax.experimental import pallas as pl

_H = 224
_W = 224


def _bf(v):
    return float(np.asarray(v, dtype=jnp.bfloat16).astype(np.float32))


def _gauss2d():
    v = [math.exp(-(i * i) / 2.0) for i in (-2, -1, 0, 1, 2)]
    g = np.outer(np.asarray(v, np.float32), np.asarray(v, np.float32))
    g = (g / g.sum()).astype(np.float32)
    return [[_bf(g[i, j]) for j in range(5)] for i in range(5)]


_GK = _gauss2d()


def _sh(a, d):
    if d == 0:
        return a
    z = jnp.zeros((abs(d), a.shape[1]), a.dtype)
    if d > 0:
        return jnp.concatenate([a[d:], z], axis=0)
    return jnp.concatenate([z, a[:d]], axis=0)


def _sw(a, d):
    if d == 0:
        return a
    z = jnp.zeros((a.shape[0], abs(d)), a.dtype)
    if d > 0:
        return jnp.concatenate([a[:, d:], z], axis=1)
    return jnp.concatenate([z, a[:, :d]], axis=1)


def _conv5_bf16(a):
    ab = a.astype(jnp.bfloat16).astype(jnp.float32)
    shs = [_sh(ab, k) for k in (-2, -1, 0, 1, 2)]
    acc = None
    for j in range(5):
        cj = _GK[0][j] * shs[0]
        for i in range(1, 5):
            cj = cj + _GK[i][j] * shs[i]
        t = _sw(cj, j - 2)
        acc = t if acc is None else acc + t
    return acc


def _sobel_bf16(a):
    ab = a.astype(jnp.bfloat16).astype(jnp.float32)
    sm = _sh(ab, -1)
    sp = _sh(ab, 1)
    t1 = sm + 2.0 * ab + sp
    gx = _sw(t1, 1) - _sw(t1, -1)
    t2 = sp - sm
    gy = _sw(t2, -1) + 2.0 * t2 + _sw(t2, 1)
    return gx, gy


def _canny_body(x_ref, o_ref):
    x0 = x_ref[0, 0]
    x1 = x_ref[0, 1]
    x2 = x_ref[0, 2]
    g = 0.299 * x0 + 0.587 * x1 + 0.114 * x2

    s = _conv5_bf16(_conv5_bf16(g))
    gx, gy = _sobel_bf16(s)

    mag = jnp.sqrt(gx * gx + gy * gy)

    ax = jnp.abs(gx)
    ay = jnp.abs(gy)
    c0 = ay <= 0.41421356237309503 * ax
    c90 = ay >= 2.414213562373095 * ax
    c45 = jnp.logical_and(jnp.logical_not(c0), jnp.logical_not(c90))
    c45 = jnp.logical_and(c45, gx * gy > 0)

    swm = _sw(mag, -1)
    swp = _sw(mag, 1)
    u = _sh(mag, -1)
    d = _sh(mag, 1)
    ul = _sh(swm, -1)
    dl = _sh(swm, 1)
    ur = _sh(swp, -1)
    dr = _sh(swp, 1)

    nmax = jnp.where(
        c0, jnp.maximum(swm, swp),
        jnp.where(c45, jnp.maximum(ur, dl),
                  jnp.where(c90, jnp.maximum(u, d), jnp.maximum(ul, dr))))

    iy = lax.broadcasted_iota(jnp.int32, (_H, _W), 0)
    ix = lax.broadcasted_iota(jnp.int32, (_H, _W), 1)
    interior = jnp.logical_and(
        jnp.logical_and(iy >= 1, iy <= _H - 2),
        jnp.logical_and(ix >= 1, ix <= _W - 2))

    keep = jnp.logical_and(interior, mag >= nmax)
    nms = jnp.where(keep, mag, 0.0)
    lowm = nms > 0.1
    edge0 = jnp.where(nms > 0.2, 1.0, 0.0)

    def cond(c):
        return c[1]

    def body(c):
        e, _ = c
        m1 = jnp.maximum(jnp.maximum(_sh(e, -1), e), _sh(e, 1))
        dil = jnp.maximum(jnp.maximum(_sw(m1, -1), m1), _sw(m1, 1))
        new = jnp.where(jnp.logical_and(lowm, dil > 0.0), 1.0, e)
        return new, jnp.max(new - e) > 0.0

    edge, _ = lax.while_loop(cond, body, (edge0, jnp.bool_(True)))

    o_ref[0, 0] = 1.0 - edge


def _build_call(interpret=False):
    return pl.pallas_call(
        _canny_body,
        grid=(2,),
        in_specs=[pl.BlockSpec((1, 3, _H, _W), lambda b: (b, 0, 0, 0))],
        out_specs=pl.BlockSpec((1, 1, _H, _W), lambda b: (b, 0, 0, 0)),
        out_shape=jax.ShapeDtypeStruct((2, 1, _H, _W), jnp.float32),
        interpret=interpret,
    )


@jax.jit
def kernel(x):
    return _build_call()(x)
